# Initial kernel scaffold; baseline (speedup 1.0000x reference)
import functools

import jax
import jax.numpy as jnp
from jax import lax
from jax.experimental import pallas as pl
from jax.experimental.pallas import tpu as pltpu
from jax.experimental.pallas import tpu_sc as plsc

_INFO = plsc.get_sparse_core_info()
_NC, _NS = _INFO.num_cores, _INFO.num_subcores
_NW = _NC * _NS

_CHUNK = 128


@functools.partial(jax.jit, static_argnames=("n_flat", "dim"))
def _lookup(idx_flat, w, *, n_flat, dim):
    b_per_w = n_flat // _NW
    n_chunks = b_per_w // _CHUNK
    mesh = plsc.VectorSubcoreMesh(core_axis_name="c", subcore_axis_name="s")

    @functools.partial(
        pl.kernel,
        out_type=jax.ShapeDtypeStruct((n_flat, dim), jnp.float32),
        mesh=mesh,
        scratch_types=[
            pltpu.VMEM((_CHUNK,), jnp.int32),
            pltpu.VMEM((_CHUNK, dim), jnp.float32),
            pltpu.SemaphoreType.DMA,
        ],
    )
    def body(table_hbm, idx_hbm, out_hbm, idx_v, rows_v, sem):
        wid = lax.axis_index("s") * _NC + lax.axis_index("c")
        base = wid * b_per_w

        def step(i, _):
            off = base + i * _CHUNK
            pltpu.sync_copy(idx_hbm.at[pl.ds(off, _CHUNK)], idx_v)
            pltpu.async_copy(table_hbm.at[idx_v], rows_v, sem).wait()
            pltpu.sync_copy(rows_v, out_hbm.at[pl.ds(off, _CHUNK)])
            return ()

        lax.fori_loop(0, n_chunks, step, (), unroll=False)

    return body(w, idx_flat)


def kernel(token_ids, w):
    b, t = token_ids.shape
    n_flat = b * t
    idx_flat = token_ids.reshape(n_flat).astype(jnp.int32)
    out = _lookup(idx_flat, w, n_flat=n_flat, dim=w.shape[1])
    return out.reshape(b, t, w.shape[1])


# speedup vs baseline: 1.5746x; 1.5746x over previous
import functools

import jax
import jax.numpy as jnp
from jax import lax
from jax.experimental import pallas as pl
from jax.experimental.pallas import tpu as pltpu
from jax.experimental.pallas import tpu_sc as plsc

_INFO = plsc.get_sparse_core_info()
_NC, _NS = _INFO.num_cores, _INFO.num_subcores
_NW = _NC * _NS

_CHUNK = 128


@functools.partial(jax.jit, static_argnames=("n_flat", "dim"))
def _lookup(idx_flat, w, *, n_flat, dim):
    b_per_w = n_flat // _NW
    n_chunks = b_per_w // _CHUNK
    mesh = plsc.VectorSubcoreMesh(core_axis_name="c", subcore_axis_name="s")

    @functools.partial(
        pl.kernel,
        out_type=jax.ShapeDtypeStruct((n_flat, dim), jnp.float32),
        mesh=mesh,
        scratch_types=[
            pltpu.VMEM((_CHUNK,), jnp.int32),
            pltpu.VMEM((_CHUNK, dim), jnp.float32),
            pltpu.SemaphoreType.DMA,
        ],
        compiler_params=pltpu.CompilerParams(use_tc_tiling_on_sc=False),
    )
    def body(table_hbm, idx_hbm, out_hbm, idx_v, rows_v, sem):
        wid = lax.axis_index("s") * _NC + lax.axis_index("c")
        base = wid * b_per_w

        def step(i, _):
            off = base + i * _CHUNK
            pltpu.sync_copy(idx_hbm.at[pl.ds(off, _CHUNK)], idx_v)
            pltpu.async_copy(table_hbm.at[idx_v], rows_v, sem).wait()
            pltpu.sync_copy(rows_v, out_hbm.at[pl.ds(off, _CHUNK)])
            return ()

        lax.fori_loop(0, n_chunks, step, (), unroll=False)

    return body(w, idx_flat)


def kernel(token_ids, w):
    b, t = token_ids.shape
    n_flat = b * t
    idx_flat = token_ids.reshape(n_flat).astype(jnp.int32)
    out = _lookup(idx_flat, w, n_flat=n_flat, dim=w.shape[1])
    return out.reshape(b, t, w.shape[1])


# speedup vs baseline: 1.8717x; 1.1887x over previous
import functools

import jax
import jax.numpy as jnp
from jax import lax
from jax.experimental import pallas as pl
from jax.experimental.pallas import tpu as pltpu
from jax.experimental.pallas import tpu_sc as plsc

_INFO = plsc.get_sparse_core_info()
_NC, _NS = _INFO.num_cores, _INFO.num_subcores
_NW = _NC * _NS

_CHUNK = 128
_NBUF = 8


@functools.partial(jax.jit, static_argnames=("n_flat", "dim"))
def _lookup(idx_grouped, w, *, n_flat, dim):
    b_per_w = n_flat // _NW
    n_chunks = b_per_w // _CHUNK
    n_rounds = n_chunks // _NBUF
    mesh = plsc.VectorSubcoreMesh(core_axis_name="c", subcore_axis_name="s")

    @functools.partial(
        pl.kernel,
        out_type=jax.ShapeDtypeStruct((n_flat, dim), jnp.float32),
        mesh=mesh,
        scratch_types=[
            pltpu.VMEM((n_chunks, _CHUNK), jnp.int32),
            pltpu.VMEM((_NBUF, _CHUNK, dim), jnp.float32),
            pltpu.SemaphoreType.DMA((_NBUF,)),
            pltpu.SemaphoreType.DMA((_NBUF,)),
        ],
        compiler_params=pltpu.CompilerParams(use_tc_tiling_on_sc=False),
    )
    def body(table_hbm, idx_hbm, out_hbm, idx_v, rows_v, gsem, ssem):
        wid = lax.axis_index("s") * _NC + lax.axis_index("c")
        base = wid * b_per_w

        pltpu.sync_copy(idx_hbm.at[wid], idx_v)

        def start_gather(c, b):
            pltpu.make_async_copy(
                table_hbm.at[idx_v.at[c]], rows_v.at[b], gsem.at[b]
            ).start()

        def wait_gather(b):
            pltpu.make_async_copy(
                table_hbm.at[idx_v.at[0]], rows_v.at[b], gsem.at[b]
            ).wait()

        def start_store(c, b):
            pltpu.make_async_copy(
                rows_v.at[b], out_hbm.at[pl.ds(base + c * _CHUNK, _CHUNK)],
                ssem.at[b],
            ).start()

        def wait_store(b):
            pltpu.make_async_copy(
                rows_v.at[b], out_hbm.at[pl.ds(base, _CHUNK)], ssem.at[b]
            ).wait()

        for b in range(_NBUF):
            start_gather(b, b)

        def round_body(r, _):
            for b in range(_NBUF):
                wait_gather(b)
                start_store(r * _NBUF + b, b)
            for b in range(_NBUF):
                wait_store(b)

                @pl.when(r + 1 < n_rounds)
                def _():
                    start_gather((r + 1) * _NBUF + b, b)

            return ()

        lax.fori_loop(0, n_rounds, round_body, (), unroll=False)

    return body(w, idx_grouped)


def kernel(token_ids, w):
    b, t = token_ids.shape
    n_flat = b * t
    b_per_w = n_flat // _NW
    idx_grouped = token_ids.reshape(_NW, b_per_w // _CHUNK, _CHUNK).astype(
        jnp.int32
    )
    out = _lookup(idx_grouped, w, n_flat=n_flat, dim=w.shape[1])
    return out.reshape(b, t, w.shape[1])


# speedup vs baseline: 1.8745x; 1.0015x over previous
import functools

import jax
import jax.numpy as jnp
from jax import lax
from jax.experimental import pallas as pl
from jax.experimental.pallas import tpu as pltpu
from jax.experimental.pallas import tpu_sc as plsc

_INFO = plsc.get_sparse_core_info()
_NC, _NS = _INFO.num_cores, _INFO.num_subcores
_NW = _NC * _NS

_CHUNK = 128
_NBUF = 10


@functools.partial(jax.jit, static_argnames=("n_flat", "dim"))
def _lookup(idx_grouped, w, *, n_flat, dim):
    b_per_w = n_flat // _NW
    n_chunks = b_per_w // _CHUNK
    assert b_per_w % _CHUNK == 0 and n_chunks % _NBUF == 0
    n_rounds = n_chunks // _NBUF
    mesh = plsc.VectorSubcoreMesh(core_axis_name="c", subcore_axis_name="s")

    @functools.partial(
        pl.kernel,
        out_type=jax.ShapeDtypeStruct((n_flat, dim), jnp.float32),
        mesh=mesh,
        scratch_types=[
            pltpu.VMEM((n_chunks, _CHUNK), jnp.int32),
            pltpu.VMEM((_NBUF, _CHUNK, dim), jnp.float32),
            pltpu.SemaphoreType.DMA((_NBUF,)),
            pltpu.SemaphoreType.DMA((_NBUF,)),
        ],
        compiler_params=pltpu.CompilerParams(use_tc_tiling_on_sc=False),
    )
    def body(table_hbm, idx_hbm, out_hbm, idx_v, rows_v, gsem, ssem):
        wid = lax.axis_index("s") * _NC + lax.axis_index("c")
        base = wid * b_per_w

        pltpu.sync_copy(idx_hbm.at[wid], idx_v)

        def start_gather(c, b):
            pltpu.make_async_copy(
                table_hbm.at[idx_v.at[c]], rows_v.at[b], gsem.at[b]
            ).start()

        def wait_gather(b):
            pltpu.make_async_copy(
                table_hbm.at[idx_v.at[0]], rows_v.at[b], gsem.at[b]
            ).wait()

        def start_store(c, b):
            pltpu.make_async_copy(
                rows_v.at[b], out_hbm.at[pl.ds(base + c * _CHUNK, _CHUNK)],
                ssem.at[b],
            ).start()

        def wait_store(b):
            pltpu.make_async_copy(
                rows_v.at[b], out_hbm.at[pl.ds(base, _CHUNK)], ssem.at[b]
            ).wait()

        for b in range(_NBUF):
            start_gather(b, b)

        def round_body(r, _):
            for b in range(_NBUF):
                wait_gather(b)
                start_store(r * _NBUF + b, b)
            for b in range(_NBUF):
                wait_store(b)

                @pl.when(r + 1 < n_rounds)
                def _():
                    start_gather((r + 1) * _NBUF + b, b)

            return ()

        lax.fori_loop(0, n_rounds, round_body, (), unroll=False)

    return body(w, idx_grouped)


def kernel(token_ids, w):
    b, t = token_ids.shape
    n_flat = b * t
    b_per_w = n_flat // _NW
    idx_grouped = token_ids.reshape(_NW, b_per_w // _CHUNK, _CHUNK).astype(
        jnp.int32
    )
    out = _lookup(idx_grouped, w, n_flat=n_flat, dim=w.shape[1])
    return out.reshape(b, t, w.shape[1])
